# baseline (speedup 1.0000x reference)
import functools
import math

import jax
import jax.numpy as jnp
from jax import lax
from jax.experimental import pallas as pl
from jax.experimental.pallas import tpu as pltpu
from jax.experimental.pallas import tpu_sc as plsc

_N_JOINTS = 55
_EXP_DIM = 100
_FREE_JOINTS = (0, 12, 15, 16, 17, 22, 23, 24)
_NJ = len(_FREE_JOINTS)

_COS_C = tuple((-1.0) ** k / math.factorial(2 * k) for k in range(8))
_SINC_C = tuple((-1.0) ** k / math.factorial(2 * k + 1) for k in range(7))
_VERC_C = tuple((-1.0) ** k / math.factorial(2 * k + 2) for k in range(7))


def _horner(u, coeffs):
    acc = jnp.full((16,), coeffs[-1], jnp.float32)
    for c in coeffs[-2::-1]:
        acc = acc * u + c
    return acc


def _sc_body(nc, bpw,
             pose_hbm, exp_hbm, cam_hbm, light_hbm, ids_hbm,
             out_mats, out_cam, out_exp, out_light,
             idx_v, pidx, eidx, cidx, lidx,
             pose_d, exp_d, cam_d, light_d, mats_v,
             sem_p, sem_e, sem_c, sem_l):
    wid = lax.axis_index("s") * nc + lax.axis_index("c")
    base = wid * bpw

    pltpu.sync_copy(ids_hbm.at[pl.ds(base, bpw)], idx_v)
    lanes = lax.iota(jnp.int32, 16)

    def gen(g, carry):
        f16 = idx_v[pl.ds(g * 16, 16)]
        pos = g * 16 + lanes
        pb = f16 * (3 * _NJ)
        for o in range(3 * _NJ):
            plsc.store_scatter(pidx, [pos * (3 * _NJ) + o], pb + o)
        eb = f16 * _EXP_DIM
        for o in range(_EXP_DIM):
            plsc.store_scatter(eidx, [pos * _EXP_DIM + o], eb + o)
        cb = f16 * 3
        for o in range(3):
            plsc.store_scatter(cidx, [pos * 3 + o], cb + o)
        lb = f16 * 27
        for o in range(27):
            plsc.store_scatter(lidx, [pos * 27 + o], lb + o)
        return carry

    lax.fori_loop(0, bpw // 16, gen, 0)

    g_pose = pltpu.async_copy(pose_hbm.at[pidx], pose_d, sem_p)
    g_exp = pltpu.async_copy(exp_hbm.at[eidx], exp_d, sem_e)
    g_cam = pltpu.async_copy(cam_hbm.at[cidx], cam_d, sem_c)
    g_light = pltpu.async_copy(light_hbm.at[lidx], light_d, sem_l)

    g_pose.wait()

    def group(g, carry):
        frames = g * 16 + lanes
        fb = frames * (3 * _NJ)
        for k in range(_NJ):
            x = plsc.load_gather(pose_d, [fb + (3 * k)])
            y = plsc.load_gather(pose_d, [fb + (3 * k + 1)])
            z = plsc.load_gather(pose_d, [fb + (3 * k + 2)])
            u = x * x + y * y + z * z
            c = _horner(u, _COS_C)
            a = _horner(u, _SINC_C)
            v = _horner(u, _VERC_C)
            ax, ay, az = a * x, a * y, a * z
            vx, vy, vz = v * x, v * y, v * z
            ent = (
                c + vx * x, vx * y - az, vx * z + ay,
                vx * y + az, c + vy * y, vy * z - ax,
                vx * z - ay, vy * z + ax, c + vz * z,
            )
            for e, val in enumerate(ent):
                mats_v[k * 9 + e, pl.ds(g * 16, 16)] = val
        return carry

    lax.fori_loop(0, bpw // 16, group, 0)
    pltpu.sync_copy(mats_v, out_mats.at[wid])

    g_exp.wait()
    pltpu.sync_copy(exp_d, out_exp.at[wid])
    g_cam.wait()
    pltpu.sync_copy(cam_d, out_cam.at[wid])
    g_light.wait()
    pltpu.sync_copy(light_d, out_light.at[wid])


def _tc_merge_body(fp_ref, mats_ref, out_ref):
    out_ref[...] = fp_ref[...]
    for k, j in enumerate(_FREE_JOINTS):
        for e in range(9):
            out_ref[e, j, :] = mats_ref[0, k * 9 + e, :]


def kernel(pose_table, exp_table, cam_table, light_table, full_pose, frame_ids):
    b = frame_ids.shape[0]
    n_frames = pose_table.shape[0]
    info = plsc.get_sparse_core_info()
    nc, ns = info.num_cores, info.num_subcores
    nw = nc * ns
    bpw = b // nw

    pose_flat = pose_table[:, _FREE_JOINTS, :].reshape(-1)
    exp_flat = exp_table.reshape(-1)
    cam_flat = cam_table.reshape(-1)
    light_flat = light_table.reshape(-1)

    mesh = plsc.VectorSubcoreMesh(core_axis_name="c", subcore_axis_name="s")
    run = functools.partial(
        pl.kernel,
        out_type=(
            jax.ShapeDtypeStruct((nw, 72, bpw), jnp.float32),
            jax.ShapeDtypeStruct((nw, bpw * 3), jnp.float32),
            jax.ShapeDtypeStruct((nw, bpw * _EXP_DIM), jnp.float32),
            jax.ShapeDtypeStruct((nw, bpw * 27), jnp.float32),
        ),
        mesh=mesh,
        compiler_params=pltpu.CompilerParams(
            needs_layout_passes=False, use_tc_tiling_on_sc=False),
        scratch_types=(
            pltpu.VMEM((bpw,), jnp.int32),
            pltpu.VMEM((bpw * 3 * _NJ,), jnp.int32),
            pltpu.VMEM((bpw * _EXP_DIM,), jnp.int32),
            pltpu.VMEM((bpw * 3,), jnp.int32),
            pltpu.VMEM((bpw * 27,), jnp.int32),
            pltpu.VMEM((bpw * 3 * _NJ,), jnp.float32),
            pltpu.VMEM((bpw * _EXP_DIM,), jnp.float32),
            pltpu.VMEM((bpw * 3,), jnp.float32),
            pltpu.VMEM((bpw * 27,), jnp.float32),
            pltpu.VMEM((72, bpw), jnp.float32),
            pltpu.SemaphoreType.DMA,
            pltpu.SemaphoreType.DMA,
            pltpu.SemaphoreType.DMA,
            pltpu.SemaphoreType.DMA,
        ),
    )(functools.partial(_sc_body, nc, bpw))

    mats, cam, exp, light = run(
        pose_flat, exp_flat, cam_flat, light_flat,
        frame_ids.astype(jnp.int32))

    fp_t = jnp.transpose(full_pose, (2, 3, 1, 0)).reshape(9, _N_JOINTS, b)
    out_t = pl.pallas_call(
        _tc_merge_body,
        grid=(nw,),
        in_specs=[
            pl.BlockSpec((9, _N_JOINTS, bpw), lambda i: (0, 0, i)),
            pl.BlockSpec((1, 72, bpw), lambda i: (i, 0, 0)),
        ],
        out_specs=pl.BlockSpec((9, _N_JOINTS, bpw), lambda i: (0, 0, i)),
        out_shape=jax.ShapeDtypeStruct((9, _N_JOINTS, b), jnp.float32),
    )(fp_t, mats)
    out_pose = jnp.transpose(
        out_t.reshape(3, 3, _N_JOINTS, b), (3, 2, 0, 1))

    return (out_pose,
            cam.reshape(b, 3),
            exp.reshape(b, _EXP_DIM),
            light.reshape(b, 9, 3))


# speedup vs baseline: 11.6935x; 11.6935x over previous
import functools
import math

import jax
import jax.numpy as jnp
from jax import lax
from jax.experimental import pallas as pl
from jax.experimental.pallas import tpu as pltpu
from jax.experimental.pallas import tpu_sc as plsc

_N_JOINTS = 55
_EXP_DIM = 100
_EW = 104
_CW = 64
_FREE_JOINTS = (0, 12, 15, 16, 17, 22, 23, 24)
_NJ = len(_FREE_JOINTS)

_COS_C = tuple((-1.0) ** k / math.factorial(2 * k) for k in range(8))
_SINC_C = tuple((-1.0) ** k / math.factorial(2 * k + 1) for k in range(7))
_VERC_C = tuple((-1.0) ** k / math.factorial(2 * k + 2) for k in range(7))


def _horner(u, coeffs):
    acc = jnp.full((16,), coeffs[-1], jnp.float32)
    for c in coeffs[-2::-1]:
        acc = acc * u + c
    return acc


def _sc_body(nc, bpw,
             combo_hbm, exp_hbm, ids_hbm,
             out_mats, out_cam, out_exp, out_light,
             idx_v, combo_v, exp_v, mats_v,
             sem_cb, sem_e):
    wid = lax.axis_index("s") * nc + lax.axis_index("c")
    base = wid * bpw

    pltpu.sync_copy(ids_hbm.at[pl.ds(base, bpw)], idx_v)
    g_combo = pltpu.async_copy(combo_hbm.at[idx_v], combo_v, sem_cb)
    g_exp = pltpu.async_copy(exp_hbm.at[idx_v], exp_v, sem_e)

    g_combo.wait()
    lanes = lax.iota(jnp.int32, 16)

    def group(g, carry):
        frames = g * 16 + lanes
        for k in range(_NJ):
            x = plsc.load_gather(combo_v, [frames, jnp.full((16,), 3 * k, jnp.int32)])
            y = plsc.load_gather(combo_v, [frames, jnp.full((16,), 3 * k + 1, jnp.int32)])
            z = plsc.load_gather(combo_v, [frames, jnp.full((16,), 3 * k + 2, jnp.int32)])
            u = x * x + y * y + z * z
            c = _horner(u, _COS_C)
            a = _horner(u, _SINC_C)
            v = _horner(u, _VERC_C)
            ax, ay, az = a * x, a * y, a * z
            vx, vy, vz = v * x, v * y, v * z
            ent = (
                c + vx * x, vx * y - az, vx * z + ay,
                vx * y + az, c + vy * y, vy * z - ax,
                vx * z - ay, vy * z + ax, c + vz * z,
            )
            for e, val in enumerate(ent):
                mats_v[k * 9 + e, pl.ds(g * 16, 16)] = val
        return carry

    lax.fori_loop(0, bpw // 16, group, 0)
    pltpu.sync_copy(mats_v, out_mats.at[wid])

    pltpu.sync_copy(combo_v.at[:, pl.ds(24, 8)],
                    out_cam.at[pl.ds(base, bpw)])
    pltpu.sync_copy(combo_v.at[:, pl.ds(32, 32)],
                    out_light.at[pl.ds(base, bpw)])
    g_exp.wait()
    pltpu.sync_copy(exp_v, out_exp.at[pl.ds(base, bpw)])


def _tc_merge_body(fp_ref, mats_ref, out_ref):
    out_ref[...] = fp_ref[...]
    for k, j in enumerate(_FREE_JOINTS):
        for e in range(9):
            out_ref[e, j, :] = mats_ref[0, k * 9 + e, :]


def kernel(pose_table, exp_table, cam_table, light_table, full_pose, frame_ids):
    b = frame_ids.shape[0]
    n_frames = pose_table.shape[0]
    info = plsc.get_sparse_core_info()
    nc, ns = info.num_cores, info.num_subcores
    nw = nc * ns
    bpw = b // nw

    zeros5 = jnp.zeros((n_frames, 5), jnp.float32)
    combo = jnp.concatenate(
        [pose_table[:, _FREE_JOINTS, :].reshape(n_frames, 3 * _NJ),
         cam_table, zeros5,
         light_table.reshape(n_frames, 27), zeros5],
        axis=1)
    exp2d = jnp.pad(exp_table, ((0, 0), (0, _EW - _EXP_DIM)))

    mesh = plsc.VectorSubcoreMesh(core_axis_name="c", subcore_axis_name="s")
    run = functools.partial(
        pl.kernel,
        out_type=(
            jax.ShapeDtypeStruct((nw, 72, bpw), jnp.float32),
            jax.ShapeDtypeStruct((b, 8), jnp.float32),
            jax.ShapeDtypeStruct((b, _EW), jnp.float32),
            jax.ShapeDtypeStruct((b, 32), jnp.float32),
        ),
        mesh=mesh,
        compiler_params=pltpu.CompilerParams(
            needs_layout_passes=False, use_tc_tiling_on_sc=False),
        scratch_types=(
            pltpu.VMEM((bpw,), jnp.int32),
            pltpu.VMEM((bpw, _CW), jnp.float32),
            pltpu.VMEM((bpw, _EW), jnp.float32),
            pltpu.VMEM((72, bpw), jnp.float32),
            pltpu.SemaphoreType.DMA,
            pltpu.SemaphoreType.DMA,
        ),
    )(functools.partial(_sc_body, nc, bpw))

    mats, cam, exp, light = run(combo, exp2d, frame_ids.astype(jnp.int32))

    fp_t = jnp.transpose(full_pose, (2, 3, 1, 0)).reshape(9, _N_JOINTS, b)
    out_t = pl.pallas_call(
        _tc_merge_body,
        grid=(nw,),
        in_specs=[
            pl.BlockSpec((9, _N_JOINTS, bpw), lambda i: (0, 0, i)),
            pl.BlockSpec((1, 72, bpw), lambda i: (i, 0, 0)),
        ],
        out_specs=pl.BlockSpec((9, _N_JOINTS, bpw), lambda i: (0, 0, i)),
        out_shape=jax.ShapeDtypeStruct((9, _N_JOINTS, b), jnp.float32),
    )(fp_t, mats)
    out_pose = jnp.transpose(
        out_t.reshape(3, 3, _N_JOINTS, b), (3, 2, 0, 1))

    return (out_pose, cam[:, :3], exp[:, :_EXP_DIM],
            light[:, :27].reshape(b, 9, 3))
